# baseline (speedup 1.0000x reference)
import jax
import jax.numpy as jnp
from jax.experimental import pallas as pl
from jax.experimental.pallas import tpu as pltpu

DIM = 1024
N_WORK = 32
N_PERSIST = 64
N_LONG = 256
N_TOT = N_WORK + N_PERSIST + N_LONG
M_PAD = 384
TILE = 512


def _body(q_ref, wc_ref, bc_ref, cmt_ref, cm_ref, o_ref):
    q = q_ref[...].astype(jnp.bfloat16)
    qp = jnp.dot(q, wc_ref[...], preferred_element_type=jnp.float32)
    qp = qp + bc_ref[...]
    s = jnp.dot(qp.astype(jnp.bfloat16), cmt_ref[...],
                preferred_element_type=jnp.float32)

    col = jax.lax.broadcasted_iota(jnp.int32, (1, M_PAD), 1)
    m0 = col < N_WORK
    m1 = (col >= N_WORK) & (col < N_WORK + N_PERSIST)
    m2 = (col >= N_WORK + N_PERSIST) & (col < N_TOT)
    neg = jnp.float32(-jnp.inf)
    mx0 = jnp.max(jnp.where(m0, s, neg), axis=-1, keepdims=True)
    mx1 = jnp.max(jnp.where(m1, s, neg), axis=-1, keepdims=True)
    mx2 = jnp.max(jnp.where(m2, s, neg), axis=-1, keepdims=True)
    mx_sel = jnp.where(m0, mx0, jnp.where(m1, mx1, mx2))
    e = jnp.where(col < N_TOT, jnp.exp(s - mx_sel), 0.0)
    d0 = jnp.sum(jnp.where(m0, e, 0.0), axis=-1, keepdims=True)
    d1 = jnp.sum(jnp.where(m1, e, 0.0), axis=-1, keepdims=True)
    d2 = jnp.sum(jnp.where(m2, e, 0.0), axis=-1, keepdims=True)
    third = jnp.float32(1.0 / 3.0)
    r = jnp.where(m0, third / d0, jnp.where(m1, third / d1, third / d2))
    probs = e * r
    o_ref[...] = jnp.dot(probs.astype(jnp.bfloat16), cm_ref[...],
                         preferred_element_type=jnp.float32)


@jax.jit
def kernel(query_states, Wc, bc, working_memory, persistent_memory,
           long_term_buffer):
    B, S, D = query_states.shape
    q2 = query_states.reshape(B * S, D)
    cmem = jnp.concatenate(
        [working_memory[0], persistent_memory[0], long_term_buffer[0],
         jnp.zeros((M_PAD - N_TOT, D), dtype=query_states.dtype)], axis=0)
    scale = 1.0 / jnp.sqrt(jnp.float32(DIM))
    cmt = (cmem.T * scale).astype(jnp.bfloat16)
    cm16 = cmem.astype(jnp.bfloat16)
    wc16 = Wc.astype(jnp.bfloat16)
    bc2 = bc.reshape(1, D)

    grid = (B * S // TILE,)
    out = pl.pallas_call(
        _body,
        grid=grid,
        in_specs=[
            pl.BlockSpec((TILE, D), lambda i: (i, 0)),
            pl.BlockSpec((D, D), lambda i: (0, 0)),
            pl.BlockSpec((1, D), lambda i: (0, 0)),
            pl.BlockSpec((D, M_PAD), lambda i: (0, 0)),
            pl.BlockSpec((M_PAD, D), lambda i: (0, 0)),
        ],
        out_specs=pl.BlockSpec((TILE, D), lambda i: (i, 0)),
        out_shape=jax.ShapeDtypeStruct((B * S, D), jnp.float32),
        compiler_params=pltpu.CompilerParams(
            dimension_semantics=("parallel",)),
    )(q2, wc16, bc2, cmt, cm16)
    return out.reshape(B, S, D)
